# baseline (speedup 1.0000x reference)
import functools

import jax
import jax.numpy as jnp
from jax import lax
from jax.experimental import pallas as pl
from jax.experimental.pallas import tpu as pltpu
from jax.experimental.pallas import tpu_sc as plsc

D_MODEL = 64
ROWS_PER_CHUNK = 128
SCALE = 64.0


@functools.partial(jax.jit, static_argnames=("n_chunks_per_worker",))
def _run(x2d, lut, n_chunks_per_worker):
    info = plsc.get_sparse_core_info()
    nc, ns = info.num_cores, info.num_subcores
    n_rows = x2d.shape[0] * ROWS_PER_CHUNK
    mesh = plsc.VectorSubcoreMesh(core_axis_name="c", subcore_axis_name="s")

    @functools.partial(
        pl.kernel,
        mesh=mesh,
        out_type=jax.ShapeDtypeStruct((n_rows, D_MODEL), jnp.float32),
        scratch_types=[
            pltpu.VMEM((n_chunks_per_worker, ROWS_PER_CHUNK), jnp.int32),
            pltpu.VMEM((ROWS_PER_CHUNK, D_MODEL), jnp.float32),
            pltpu.SemaphoreType.DMA,
        ],
        compiler_params=pltpu.CompilerParams(use_tc_tiling_on_sc=False),
    )
    def k(x_hbm, lut_hbm, out_hbm, idx_v, rows_v, sem):
        wid = lax.axis_index("s") * nc + lax.axis_index("c")
        cbase = wid * n_chunks_per_worker
        pltpu.sync_copy(x_hbm.at[pl.ds(cbase, n_chunks_per_worker)], idx_v)

        def chunk(j, carry):
            pltpu.async_copy(lut_hbm.at[idx_v.at[j]], rows_v, sem).wait()

            def mulrow(r, c2):
                for c in range(D_MODEL // 16):
                    sl = pl.ds(c * 16, 16)
                    rows_v[r, sl] = rows_v[r, sl] * SCALE
                return c2

            lax.fori_loop(0, ROWS_PER_CHUNK, mulrow, 0)
            pltpu.sync_copy(
                rows_v,
                out_hbm.at[pl.ds((cbase + j) * ROWS_PER_CHUNK, ROWS_PER_CHUNK)],
            )
            return carry

        lax.fori_loop(0, n_chunks_per_worker, chunk, 0)

    return k(x2d, lut)


def kernel(x, lut):
    b, s = x.shape
    n_rows = b * s
    info = plsc.get_sparse_core_info()
    n_workers = info.num_cores * info.num_subcores
    n_chunks = n_rows // ROWS_PER_CHUNK
    assert n_rows % (ROWS_PER_CHUNK * n_workers) == 0
    x2d = x.reshape(n_chunks, ROWS_PER_CHUNK).astype(jnp.int32)
    out = _run(x2d, lut, n_chunks // n_workers)
    return out.reshape(b, s, D_MODEL)


# speedup vs baseline: 1.2020x; 1.2020x over previous
import functools

import jax
import jax.numpy as jnp
from jax import lax
from jax.experimental import pallas as pl
from jax.experimental.pallas import tpu as pltpu
from jax.experimental.pallas import tpu_sc as plsc

D_MODEL = 64
ROWS_PER_CHUNK = 128
NBUF = 4
SCALE = 64.0
ROW_UNROLL = 4


@functools.partial(jax.jit, static_argnames=("n_chunks_per_worker",))
def _run(x2d, lut, n_chunks_per_worker):
    info = plsc.get_sparse_core_info()
    nc = info.num_cores
    n_rows = x2d.shape[0] * ROWS_PER_CHUNK
    n_laps = n_chunks_per_worker // NBUF
    mesh = plsc.VectorSubcoreMesh(core_axis_name="c", subcore_axis_name="s")

    @functools.partial(
        pl.kernel,
        mesh=mesh,
        out_type=jax.ShapeDtypeStruct((n_rows, D_MODEL), jnp.float32),
        scratch_types=[
            pltpu.VMEM((n_chunks_per_worker, ROWS_PER_CHUNK), jnp.int32),
            pltpu.VMEM((NBUF, ROWS_PER_CHUNK, D_MODEL), jnp.float32),
            pltpu.VMEM((NBUF, ROWS_PER_CHUNK, D_MODEL), jnp.float32),
            pltpu.SemaphoreType.DMA((NBUF,)),
            pltpu.SemaphoreType.DMA((NBUF,)),
        ],
        compiler_params=pltpu.CompilerParams(use_tc_tiling_on_sc=False),
    )
    def k(x_hbm, lut_hbm, out_hbm, idx_v, in_b, out_b, gsem, osem):
        wid = lax.axis_index("s") * nc + lax.axis_index("c")
        cbase = wid * n_chunks_per_worker
        pltpu.sync_copy(x_hbm.at[pl.ds(cbase, n_chunks_per_worker)], idx_v)

        for b in range(NBUF):
            pltpu.async_copy(lut_hbm.at[idx_v.at[b]], in_b.at[b], gsem.at[b])

        def lap(t, carry):
            for b in range(NBUF):
                j = t * NBUF + b
                pltpu.make_async_copy(
                    lut_hbm.at[idx_v.at[j]], in_b.at[b], gsem.at[b]
                ).wait()
                @pl.when(t > 0)
                def _wait_out():
                    pltpu.make_async_copy(
                        out_b.at[b],
                        out_hbm.at[pl.ds(0, ROWS_PER_CHUNK)],
                        osem.at[b],
                    ).wait()

                def mulrow(r, c2):
                    for rr in range(ROW_UNROLL):
                        row = r * ROW_UNROLL + rr
                        for c in range(D_MODEL // 16):
                            sl = pl.ds(c * 16, 16)
                            out_b[b, row, sl] = in_b[b, row, sl] * SCALE
                    return c2

                lax.fori_loop(0, ROWS_PER_CHUNK // ROW_UNROLL, mulrow, 0)
                pltpu.async_copy(
                    out_b.at[b],
                    out_hbm.at[pl.ds((cbase + j) * ROWS_PER_CHUNK, ROWS_PER_CHUNK)],
                    osem.at[b],
                )
                @pl.when(t < n_laps - 1)
                def _refill():
                    pltpu.async_copy(
                        lut_hbm.at[idx_v.at[j + NBUF]], in_b.at[b], gsem.at[b]
                    )

            return carry

        lax.fori_loop(0, n_laps, lap, 0)
        for b in range(NBUF):
            pltpu.make_async_copy(
                out_b.at[b], out_hbm.at[pl.ds(0, ROWS_PER_CHUNK)], osem.at[b]
            ).wait()

    return k(x2d, lut)


def kernel(x, lut):
    b, s = x.shape
    n_rows = b * s
    info = plsc.get_sparse_core_info()
    n_workers = info.num_cores * info.num_subcores
    n_chunks = n_rows // ROWS_PER_CHUNK
    assert n_rows % (ROWS_PER_CHUNK * n_workers) == 0
    x2d = x.reshape(n_chunks, ROWS_PER_CHUNK).astype(jnp.int32)
    out = _run(x2d, lut, n_chunks // n_workers)
    return out.reshape(b, s, D_MODEL)
